# Initial kernel scaffold; baseline (speedup 1.0000x reference)
import jax
import jax.numpy as jnp
from jax.experimental import pallas as pl


def kernel(inputs, position_ids, token_type_ids, tok_table, pos_table, type_table, gamma, beta):
    raise NotImplementedError("write your pallas kernel here")


# speedup vs baseline: 4.3142x; 4.3142x over previous
import functools

import jax
import jax.numpy as jnp
from jax import lax
from jax.experimental import pallas as pl
from jax.experimental.pallas import tpu as pltpu
from jax.experimental.pallas import tpu_sc as plsc

BATCH = 1024
SEQ = 200
D = 128
N_TOKENS = 100000
MAX_SEQ_LEN = 512
LN_EPS = 1e-12

NC = 2
NS = 16
NW = NC * NS
L = 16
KD = D // L

T = BATCH * SEQ
TPW = T // NW
C = 128
NCHUNK = TPW // C
GRP = C // L


def _rsqrt_newton(v):
    i = lax.bitcast_convert_type(v, jnp.int32)
    i = jnp.int32(0x5F3759DF) - lax.shift_right_arithmetic(i, jnp.int32(1))
    y = lax.bitcast_convert_type(i, jnp.float32)
    for _ in range(3):
        y = y * (jnp.float32(1.5) - jnp.float32(0.5) * v * y * y)
    return y


def _body(tok_ids, pos_ids, typ_ids, tok_tab, pos_tab, typ_tab, gamma, beta,
          out, idx_tok_v, idx_pos_v, idx_typ_v, pos_tab_v, typ_tab_v, gb_v,
          rows0, rows1, gsem0, gsem1):
    c = lax.axis_index("c")
    s = lax.axis_index("s")
    wid = s * NC + c
    base = wid * TPW

    pltpu.sync_copy(tok_ids.at[pl.ds(base, TPW)], idx_tok_v)
    pltpu.sync_copy(pos_ids.at[pl.ds(base, TPW)], idx_pos_v)
    pltpu.sync_copy(typ_ids.at[pl.ds(base, TPW)], idx_typ_v)
    pltpu.sync_copy(pos_tab, pos_tab_v)
    pltpu.sync_copy(typ_tab, typ_tab_v)
    pltpu.sync_copy(gamma, gb_v.at[0])
    pltpu.sync_copy(beta, gb_v.at[1])

    bufs = (rows0, rows1)
    sems = (gsem0, gsem1)
    iota = lax.iota(jnp.int32, L)

    t0 = [typ_tab_v[0, pl.ds(k * L, L)] for k in range(KD)]
    td = [typ_tab_v[1, pl.ds(k * L, L)] - t0[k] for k in range(KD)]
    gv = [gb_v[0, pl.ds(k * L, L)] for k in range(KD)]
    bv = [gb_v[1, pl.ds(k * L, L)] for k in range(KD)]

    def issue_gather(g, par):
        idx = idx_tok_v.at[pl.ds(g * C, C)]
        pltpu.async_copy(tok_tab.at[idx], bufs[par], sems[par])

    issue_gather(0, 0)

    @pl.loop(0, NCHUNK, step=2)
    def chunk_loop(g0):
        for par in range(2):
            g = g0 + par
            buf = bufs[par]
            pltpu.make_async_copy(tok_tab.at[pl.ds(0, C)], buf, sems[par]).wait()

            @pl.when(g + 1 < NCHUNK)
            def _():
                issue_gather(g + 1, 1 - par)

            @pl.loop(0, GRP)
            def grp_loop(gi):
                for j in range(L):
                    t = gi * L + j
                    toff = g * C + t
                    ov = lax.broadcast(toff, (L,))
                    pid = plsc.load_gather(idx_pos_v, [ov])
                    tid = plsc.load_gather(idx_typ_v, [ov])
                    f = tid.astype(jnp.float32)
                    x = []
                    for k in range(KD):
                        xv = buf[t, pl.ds(k * L, L)]
                        pv = plsc.load_gather(pos_tab_v, [pid, iota + (k * L)])
                        xv = xv + pv + t0[k] + f * td[k]
                        x.append(xv)
                    s1 = x[0]
                    for k in range(1, KD):
                        s1 = s1 + x[k]
                    sq = x[0] * x[0]
                    for k in range(1, KD):
                        sq = sq + x[k] * x[k]
                    tot = jnp.sum(s1)
                    tot2 = jnp.sum(sq)
                    mean = tot * jnp.float32(1.0 / D)
                    var = tot2 * jnp.float32(1.0 / D) - mean * mean
                    inv = _rsqrt_newton(var + jnp.float32(LN_EPS))
                    for k in range(KD):
                        yk = ((x[k] - mean) * inv) * gv[k] + bv[k]
                        buf[t, pl.ds(k * L, L)] = yk

            pltpu.sync_copy(buf, out.at[pl.ds(base + g * C, C)])


_sc_call = pl.kernel(
    _body,
    out_type=jax.ShapeDtypeStruct((T, D), jnp.float32),
    mesh=plsc.VectorSubcoreMesh(
        core_axis_name="c", subcore_axis_name="s", num_cores=NC,
        num_subcores=NS),
    compiler_params=pltpu.CompilerParams(needs_layout_passes=False),
    scratch_types=[
        pltpu.VMEM((TPW,), jnp.int32),
        pltpu.VMEM((TPW,), jnp.int32),
        pltpu.VMEM((TPW,), jnp.int32),
        pltpu.VMEM((MAX_SEQ_LEN, D), jnp.float32),
        pltpu.VMEM((2, D), jnp.float32),
        pltpu.VMEM((2, D), jnp.float32),
        pltpu.VMEM((C, D), jnp.float32),
        pltpu.VMEM((C, D), jnp.float32),
        pltpu.SemaphoreType.DMA,
        pltpu.SemaphoreType.DMA,
    ],
)


def kernel(inputs, position_ids, token_type_ids, tok_table, pos_table,
           type_table, gamma, beta):
    tok_ids = inputs.reshape(-1).astype(jnp.int32)
    pos_ids = position_ids.reshape(-1).astype(jnp.int32)
    typ_ids = token_type_ids.reshape(-1).astype(jnp.int32)
    out = _sc_call(tok_ids, pos_ids, typ_ids, tok_table, pos_table,
                   type_table, gamma, beta)
    return out.reshape(BATCH, SEQ, D)
